# Initial kernel scaffold; baseline (speedup 1.0000x reference)
import jax
import jax.numpy as jnp
from jax.experimental import pallas as pl


def kernel(user_id, table, W_word, b_word, W_article, b_article):
    raise NotImplementedError("write your pallas kernel here")


# speedup vs baseline: 2.2072x; 2.2072x over previous
import functools

import jax
import jax.numpy as jnp
from jax import lax
from jax.experimental import pallas as pl
from jax.experimental.pallas import tpu as pltpu
from jax.experimental.pallas import tpu_sc as plsc

_DP = 64


def _make_sc_gather(V, D, B, NC, NS):
    NW = NC * NS
    bpw = B // NW
    G = 8
    groups = bpw // G
    nfull = V // 128
    tail = V - nfull * 128
    mesh = plsc.VectorSubcoreMesh(core_axis_name="c", subcore_axis_name="s")

    @functools.partial(
        pl.kernel,
        mesh=mesh,
        out_type=jax.ShapeDtypeStruct((B, _DP), jnp.float32),
        scratch_types=[
            pltpu.VMEM((bpw + 16,), jnp.int32),
            pltpu.VMEM((G, D, 128), jnp.float32),
            pltpu.VMEM((bpw, _DP), jnp.float32),
            pltpu.SemaphoreType.DMA((G,)),
        ],
        compiler_params=pltpu.CompilerParams(
            use_tc_tiling_on_sc=True, needs_layout_passes=False),
    )
    def gk(tab, idx_hbm, out, idx_v, bufs, stg, sems):
        wid = lax.axis_index("s") * NC + lax.axis_index("c")
        base = wid * bpw
        pltpu.sync_copy(idx_hbm.at[pl.ds(base, bpw)], idx_v.at[pl.ds(0, bpw)])
        iota = lax.broadcasted_iota(jnp.int32, (16,), 0)

        def fetch(b, s):
            pltpu.async_copy(tab.at[:, pl.ds(s * 128, 128)],
                             bufs.at[b], sems.at[b])

        def group(grp, carry):
            idv = idx_v[pl.ds(grp * G, 16)]
            for b in range(G):
                fetch(b, idv[b] >> 7)
            for b in range(G):
                pltpu.make_async_copy(tab.at[:, pl.ds(0, 128)],
                                      bufs.at[b], sems.at[b]).wait()
                lvec = jnp.full((16,), 1, jnp.int32) * (idv[b] & 127)
                j = grp * G + b
                for c in range(4):
                    d_vec = jnp.minimum(iota + 16 * c, D - 1)
                    vals = plsc.load_gather(bufs.at[b], [d_vec, lvec])
                    stg[j, pl.ds(16 * c, 16)] = vals
            return carry

        lax.fori_loop(0, groups, group, 0)
        pltpu.sync_copy(stg, out.at[pl.ds(base, bpw)])

    return gk


def _make_tc_proj(B, DP, P, BB):
    grid = B // BB

    def proj_kernel(u_ref, id_ref, ww_ref, bw_ref, wa_ref, ba_ref, ow_ref, oa_ref):
        mask = (id_ref[...] != 0).astype(jnp.float32)
        x = u_ref[...] * mask
        yw = jnp.dot(x, ww_ref[...], preferred_element_type=jnp.float32) + bw_ref[...]
        ya = jnp.dot(x, wa_ref[...], preferred_element_type=jnp.float32) + ba_ref[...]
        ow_ref[...] = jnp.maximum(yw, 0.0)
        oa_ref[...] = jnp.maximum(ya, 0.0)

    return pl.pallas_call(
        proj_kernel,
        grid=(grid,),
        in_specs=[
            pl.BlockSpec((BB, DP), lambda i: (i, 0)),
            pl.BlockSpec((BB, 1), lambda i: (i, 0)),
            pl.BlockSpec((DP, P), lambda i: (0, 0)),
            pl.BlockSpec((1, P), lambda i: (0, 0)),
            pl.BlockSpec((DP, P), lambda i: (0, 0)),
            pl.BlockSpec((1, P), lambda i: (0, 0)),
        ],
        out_specs=[
            pl.BlockSpec((BB, P), lambda i: (i, 0)),
            pl.BlockSpec((BB, P), lambda i: (i, 0)),
        ],
        out_shape=[
            jax.ShapeDtypeStruct((B, P), jnp.float32),
            jax.ShapeDtypeStruct((B, P), jnp.float32),
        ],
    )


def kernel(user_id, table, W_word, b_word, W_article, b_article):
    B = user_id.shape[0]
    V, D = table.shape
    P = W_word.shape[1]
    info = plsc.get_sparse_core_info()
    NC, NS = info.num_cores, info.num_subcores

    idx = user_id.astype(jnp.int32)
    u = _make_sc_gather(V, D, B, NC, NS)(table.T, idx)
    Wwp = jnp.pad(W_word, ((0, _DP - D), (0, 0)))
    Wap = jnp.pad(W_article, ((0, _DP - D), (0, 0)))
    proj = _make_tc_proj(B, _DP, P, BB=2048)
    return tuple(
        proj(u, idx.reshape(B, 1), Wwp, b_word.reshape(1, P),
             Wap, b_article.reshape(1, P))
    )
